# baseline (speedup 1.0000x reference)
import functools

import numpy as np
import jax
import jax.numpy as jnp
from jax import lax
from jax.experimental import pallas as pl
from jax.experimental.pallas import tpu as pltpu
from jax.experimental.pallas import tpu_sc as plsc

_B, _H, _W = 2, 4096, 2048
_PLANE = _H * _W
_ROWS_PER_BLOCK = 64

_R0 = 2944
_R_SC = _H - _R0
_ROWS_PER_SUBCORE = _R_SC // 16
_GR = 8

_KS0 = np.uint32(0)
_KS1 = np.uint32(42)
_KS2 = np.uint32(0 ^ 42 ^ 0x1BD11BDA)
_KS = (_KS0, _KS1, _KS2)
_ROT = ((13, 15, 26, 6), (17, 29, 16, 24))


def _threefry_bits(c2):
    x0 = jnp.zeros_like(c2)
    x1 = c2 + _KS1
    for g in range(5):
        for r in _ROT[g % 2]:
            x0 = x0 + x1
            x1 = x0 ^ ((x1 << np.uint32(r)) | (x1 >> np.uint32(32 - r)))
        x0 = x0 + _KS[(g + 1) % 3]
        x1 = x1 + (_KS[(g + 2) % 3] + np.uint32(g + 1))
    return x0 ^ x1


def _uniform(bits):
    fb = (bits >> np.uint32(9)) | np.uint32(0x3F800000)
    return lax.bitcast_convert_type(fb, jnp.float32) - jnp.float32(1.0)


def _sampled_mul(j0, x):
    u0 = _uniform(_threefry_bits(j0))
    u1 = _uniform(_threefry_bits(j0 + np.uint32(1)))
    u2 = _uniform(_threefry_bits(j0 + np.uint32(2)))
    c01 = (u0 * u0) * u0 >= u1
    c02 = u0 >= u2
    c12 = u1 >= (u2 * u2) * u2
    x2 = x + x
    return jnp.where(c01, jnp.where(c02, jnp.zeros_like(x), x2),
                     jnp.where(c12, x, x2))


def _dropout_block(x_ref, o_ref):
    shape = x_ref.shape
    rows = lax.broadcasted_iota(jnp.int32, shape, 1)
    cols = lax.broadcasted_iota(jnp.int32, shape, 2)
    bb = lax.broadcasted_iota(jnp.int32, shape, 0)
    row0 = pl.program_id(0) * _ROWS_PER_BLOCK
    flat = bb * _PLANE + (row0 + rows) * _W + cols
    j0 = (flat * 3).astype(jnp.uint32)
    o_ref[...] = _sampled_mul(j0, x_ref[...])


def _tc_part(input):
    grid = (_R0 // _ROWS_PER_BLOCK,)
    spec = pl.BlockSpec((_B, _ROWS_PER_BLOCK, _W), lambda i: (0, i, 0))
    return pl.pallas_call(
        _dropout_block,
        grid=grid,
        in_specs=[spec],
        out_specs=spec,
        out_shape=jax.ShapeDtypeStruct((_B, _H, _W), jnp.float32),
    )(input)


def _sc_body(x_hbm, o_hbm, xrow_v, orow_v):
    b = lax.axis_index("c")
    sub = lax.axis_index("s")
    r0 = sub * _ROWS_PER_SUBCORE
    lane3 = 3 * lax.broadcasted_iota(jnp.int32, (16,), 0)

    def grp_body(g, _):
        rg = r0 + g * _GR
        pltpu.sync_copy(x_hbm.at[b, pl.ds(_R0 + rg, _GR)], xrow_v)
        for rloc in range(_GR):
            base3 = (b * _PLANE + (_R0 + rg + rloc) * _W) * 3

            @plsc.parallel_loop(0, _W, step=16)
            def chunk_body(off):
                j0 = ((base3 + off * 3) + lane3).astype(jnp.uint32)
                x = xrow_v[rloc, pl.ds(off, 16)]
                orow_v[rloc, pl.ds(off, 16)] = _sampled_mul(j0, x)
        pltpu.sync_copy(orow_v, o_hbm.at[b, pl.ds(rg, _GR)])
        return 0

    lax.fori_loop(0, _ROWS_PER_SUBCORE // _GR, grp_body, 0)


_sc_part = functools.partial(
    pl.kernel,
    mesh=plsc.VectorSubcoreMesh(core_axis_name="c", subcore_axis_name="s"),
    out_type=jax.ShapeDtypeStruct((_B, _R_SC, _W), jnp.float32),
    scratch_types=[pltpu.VMEM((_GR, _W), jnp.float32),
                   pltpu.VMEM((_GR, _W), jnp.float32)],
)(_sc_body)


def kernel(input):
    out_tc = _tc_part(input)
    out_sc = _sc_part(input)
    return lax.dynamic_update_slice(out_tc, out_sc, (0, _R0, 0))


# speedup vs baseline: 1.1106x; 1.1106x over previous
import functools

import numpy as np
import jax
import jax.numpy as jnp
from jax import lax
from jax.experimental import pallas as pl
from jax.experimental.pallas import tpu as pltpu
from jax.experimental.pallas import tpu_sc as plsc

_B, _H, _W = 2, 4096, 2048
_PLANE = _H * _W
_ROWS_PER_BLOCK = 64

_R0 = 3072
_R_SC = _H - _R0
_ROWS_PER_SUBCORE = _R_SC // 16
_GR = 8

_KS0 = np.uint32(0)
_KS1 = np.uint32(42)
_KS2 = np.uint32(0 ^ 42 ^ 0x1BD11BDA)
_KS = (_KS0, _KS1, _KS2)
_ROT = ((13, 15, 26, 6), (17, 29, 16, 24))


def _threefry_bits(c2):
    x0 = jnp.zeros_like(c2)
    x1 = c2 + _KS1
    for g in range(5):
        for r in _ROT[g % 2]:
            x0 = x0 + x1
            x1 = x0 ^ ((x1 << np.uint32(r)) | (x1 >> np.uint32(32 - r)))
        x0 = x0 + _KS[(g + 1) % 3]
        x1 = x1 + (_KS[(g + 2) % 3] + np.uint32(g + 1))
    return x0 ^ x1


def _uniform(bits):
    fb = (bits >> np.uint32(9)) | np.uint32(0x3F800000)
    return lax.bitcast_convert_type(fb, jnp.float32) - jnp.float32(1.0)


def _sampled_mul(j0, x):
    u0 = _uniform(_threefry_bits(j0))
    u1 = _uniform(_threefry_bits(j0 + np.uint32(1)))
    u2 = _uniform(_threefry_bits(j0 + np.uint32(2)))
    c01 = (u0 * u0) * u0 >= u1
    c02 = u0 >= u2
    c12 = u1 >= (u2 * u2) * u2
    x2 = x + x
    return jnp.where(c01, jnp.where(c02, jnp.zeros_like(x), x2),
                     jnp.where(c12, x, x2))


def _dropout_block(x_ref, o_ref):
    shape = x_ref.shape
    rows = lax.broadcasted_iota(jnp.int32, shape, 1)
    cols = lax.broadcasted_iota(jnp.int32, shape, 2)
    bb = lax.broadcasted_iota(jnp.int32, shape, 0)
    row0 = pl.program_id(0) * _ROWS_PER_BLOCK
    flat = bb * _PLANE + (row0 + rows) * _W + cols
    j0 = (flat * 3).astype(jnp.uint32)
    o_ref[...] = _sampled_mul(j0, x_ref[...])


def _tc_part(input):
    grid = (_R0 // _ROWS_PER_BLOCK,)
    spec = pl.BlockSpec((_B, _ROWS_PER_BLOCK, _W), lambda i: (0, i, 0))
    return pl.pallas_call(
        _dropout_block,
        grid=grid,
        in_specs=[spec],
        out_specs=spec,
        out_shape=jax.ShapeDtypeStruct((_B, _H, _W), jnp.float32),
    )(input)


def _sc_body(x_hbm, o_hbm, xrow_v, orow_v):
    b = lax.axis_index("c")
    sub = lax.axis_index("s")
    r0 = sub * _ROWS_PER_SUBCORE
    lane3 = 3 * lax.broadcasted_iota(jnp.int32, (16,), 0)

    def grp_body(g, _):
        rg = r0 + g * _GR
        pltpu.sync_copy(x_hbm.at[b, pl.ds(_R0 + rg, _GR)], xrow_v)
        for rloc in range(_GR):
            base3 = (b * _PLANE + (_R0 + rg + rloc) * _W) * 3

            @plsc.parallel_loop(0, _W, step=16)
            def chunk_body(off):
                j0 = ((base3 + off * 3) + lane3).astype(jnp.uint32)
                x = xrow_v[rloc, pl.ds(off, 16)]
                orow_v[rloc, pl.ds(off, 16)] = _sampled_mul(j0, x)
        pltpu.sync_copy(orow_v, o_hbm.at[b, pl.ds(rg, _GR)])
        return 0

    lax.fori_loop(0, _ROWS_PER_SUBCORE // _GR, grp_body, 0)


_sc_part = functools.partial(
    pl.kernel,
    mesh=plsc.VectorSubcoreMesh(core_axis_name="c", subcore_axis_name="s"),
    out_type=jax.ShapeDtypeStruct((_B, _R_SC, _W), jnp.float32),
    scratch_types=[pltpu.VMEM((_GR, _W), jnp.float32),
                   pltpu.VMEM((_GR, _W), jnp.float32)],
)(_sc_body)


def kernel(input):
    out_tc = _tc_part(input)
    out_sc = _sc_part(input)
    return lax.dynamic_update_slice(out_tc, out_sc, (0, _R0, 0))
